# baseline (speedup 1.0000x reference)
import functools

import jax
import jax.numpy as jnp
from jax import lax
from jax.experimental import pallas as pl
from jax.experimental.pallas import tpu as pltpu
from jax.experimental.pallas import tpu_sc as plsc

D = 768
L = 16
NC, NS = 2, 16
NW = NC * NS
CHUNK = 40
CHUNK_SIZES = (40, 40, 40, 40, 40, 40, 16)
CHUNK_OFFS = (0, 40, 80, 120, 160, 200, 240)


def _embed_body(word_hbm, pos_hbm, wi_hbm, pi_hbm, out_hbm,
                idxw_v, idxp_v, bufw0, bufw1, bufp0, bufp1,
                semw0, semw1, semp0, semp1, semst0, semst1):
    wid = lax.axis_index("s") * NC + lax.axis_index("c")
    rows_per_w = out_hbm.shape[0] // NW
    n_chunks = len(CHUNK_SIZES)
    base = wid * rows_per_w

    bufw = (bufw0, bufw1)
    bufp = (bufp0, bufp1)
    semw = (semw0, semw1)
    semp = (semp0, semp1)
    semst = (semst0, semst1)

    pltpu.sync_copy(wi_hbm.at[pl.ds(base, rows_per_w)], idxw_v)
    pltpu.sync_copy(pi_hbm.at[pl.ds(base, rows_per_w)], idxp_v)

    def widx(k):
        return idxw_v.at[pl.ds(CHUNK_OFFS[k], CHUNK_SIZES[k])]

    def pidx(k):
        return idxp_v.at[pl.ds(CHUNK_OFFS[k], CHUNK_SIZES[k])]

    def wdst(k, slot):
        return bufw[slot].at[pl.ds(0, CHUNK_SIZES[k])]

    def pdst(k, slot):
        return bufp[slot].at[pl.ds(0, CHUNK_SIZES[k])]

    def fire_gathers(k, slot):
        pltpu.async_copy(word_hbm.at[widx(k)], wdst(k, slot), semw[slot])
        pltpu.async_copy(pos_hbm.at[pidx(k)], pdst(k, slot), semp[slot])

    def store(k, slot):
        return pltpu.make_async_copy(
            wdst(k, slot),
            out_hbm.at[pl.ds(base + CHUNK_OFFS[k], CHUNK_SIZES[k])],
            semst[slot])

    fire_gathers(0, 0)

    for g in range(n_chunks):
        s = g % 2
        o = 1 - s
        pltpu.make_async_copy(word_hbm.at[widx(g)], wdst(g, s), semw[s]).wait()
        pltpu.make_async_copy(pos_hbm.at[pidx(g)], pdst(g, s), semp[s]).wait()
        if g >= 1:
            store(g - 1, o).wait()
        if g + 1 < n_chunks:
            fire_gathers(g + 1, o)

        def add_row(r, _, s=s):
            for c in range(D // L):
                sl = pl.ds(c * L, L)
                bufw[s][r, sl] = bufw[s][r, sl] + bufp[s][r, sl]
            return _

        lax.fori_loop(0, CHUNK_SIZES[g], add_row, 0)
        store(g, s).start()

    store(n_chunks - 1, (n_chunks - 1) % 2).wait()


@functools.partial(jax.jit, static_argnums=())
def _embed(word_embeddings, position_embeddings, wi, pi):
    n = wi.shape[0]
    rows_per_w = n // NW
    mesh = plsc.VectorSubcoreMesh(core_axis_name="c", subcore_axis_name="s",
                                  num_cores=NC, num_subcores=NS)
    return pl.kernel(
        _embed_body,
        out_type=jax.ShapeDtypeStruct((n, D), jnp.float32),
        mesh=mesh,
        scratch_types=(
            [pltpu.VMEM((rows_per_w,), jnp.int32)] * 2
            + [pltpu.VMEM((CHUNK, D), jnp.float32)] * 4
            + [pltpu.SemaphoreType.DMA] * 6
        ),
    )(word_embeddings, position_embeddings, wi, pi)


def kernel(input_ids, position_ids, word_embeddings, position_embeddings):
    B, S = input_ids.shape
    wi = input_ids.reshape(-1).astype(jnp.int32)
    pi = position_ids.reshape(-1).astype(jnp.int32)
    out = _embed(word_embeddings, position_embeddings, wi, pi)
    return out.reshape(B, S, D)
